# Initial kernel scaffold; baseline (speedup 1.0000x reference)
import jax
import jax.numpy as jnp
from jax.experimental import pallas as pl


def kernel(x_sequence, edge_index, edge_weight, Wg, bg, Wih, Whh, bih, bhh, gamma, beta):
    raise NotImplementedError("write your pallas kernel here")


# speedup vs baseline: 4.0399x; 4.0399x over previous
import functools

import jax
import jax.numpy as jnp
from jax import lax
from jax.experimental import pallas as pl
from jax.experimental.pallas import tpu as pltpu
from jax.experimental.pallas import tpu_sc as plsc

N = 10000
E = 320000
C = 128
H = 128
T = 8

NC = 2
NS = 16
NW = NC * NS
R = 320
NPAD = NW * R
K = 256
NCHUNK = (E + K - 1) // K
G = K // 16

def _wid():
    return lax.axis_index("s") * NC + lax.axis_index("c")


def _read_bounds(bounds_hbm, bv, wid):
    pltpu.sync_copy(bounds_hbm.at[wid], bv)
    b = bv[...]
    return b[0], b[1]


def _deg_body(dst_hbm, ew_hbm, bounds_hbm, deg_hbm, bv, dv, wv, acc):
    wid = _wid()
    base = wid * R
    e_lo, e_hi = _read_bounds(bounds_hbm, bv, wid)
    c_lo = e_lo // K
    c_hi = (e_hi + (K - 1)) // K

    lane = lax.iota(jnp.int32, 16)
    one0 = jnp.where(lane == 0, 1.0, 0.0)

    def zero_body(i, _):
        for u in range(8):
            acc[i, pl.ds(u * 16, 16)] = jnp.zeros((16,), jnp.float32)
        return 0
    lax.fori_loop(0, R, zero_body, 0)

    def chunk_body(c, _):
        e0 = c * K
        pltpu.sync_copy(dst_hbm.at[pl.ds(e0, K)], dv)
        pltpu.sync_copy(ew_hbm.at[pl.ds(e0, K)], wv)

        def group_body(g, _):
            p = e0 + g * 16 + lane
            d16 = dv[pl.ds(g * 16, 16)]
            w16 = wv[pl.ds(g * 16, 16)]
            valid = (p >= e_lo) & (p < e_hi)
            wm = jnp.where(valid, w16, 0.0)
            dl = jnp.clip(d16 - base, 0, R - 1)
            for i in range(16):
                plsc.addupdate(acc.at[dl[i], pl.ds(0, 16)], one0 * wm[i])
            return 0
        lax.fori_loop(0, G, group_body, 0)
        return 0
    lax.fori_loop(c_lo, c_hi, chunk_body, 0)

    def self_body(i, _):
        plsc.addupdate(acc.at[i, pl.ds(0, 16)], one0)
        return 0
    lax.fori_loop(0, R, self_body, 0)

    pltpu.sync_copy(acc, deg_hbm.at[pl.ds(base, R)])


def _msg_body(yflat_hbm, src_hbm, dst_hbm, ew_hbm, bounds_hbm, s_hbm,
              bv, sv, gv, dv, wv, rows, acc, sem):
    wid = _wid()
    base = wid * R
    e_lo, e_hi = _read_bounds(bounds_hbm, bv, wid)
    c_lo = e_lo // K
    c_hi = (e_hi + (K - 1)) // K

    lane = lax.iota(jnp.int32, 16)

    for t in range(T):
        def zero_body(i, _):
            for u in range(8):
                acc[i, pl.ds(u * 16, 16)] = jnp.zeros((16,), jnp.float32)
            return 0
        lax.fori_loop(0, R, zero_body, 0)

        def chunk_body(c, _):
            e0 = c * K
            pltpu.sync_copy(src_hbm.at[pl.ds(e0, K)], sv)
            pltpu.sync_copy(dst_hbm.at[pl.ds(e0, K)], dv)
            pltpu.sync_copy(ew_hbm.at[pl.ds(e0, K)], wv)

            def gidx_body(g, _):
                gv[pl.ds(g * 16, 16)] = sv[pl.ds(g * 16, 16)] + t * N
                return 0
            lax.fori_loop(0, G, gidx_body, 0)
            pltpu.async_copy(yflat_hbm.at[gv], rows, sem).wait()

            def group_body(g, _):
                p = e0 + g * 16 + lane
                d16 = dv[pl.ds(g * 16, 16)]
                w16 = wv[pl.ds(g * 16, 16)]
                valid = (p >= e_lo) & (p < e_hi)
                wm = jnp.where(valid, w16, 0.0)
                dl = jnp.clip(d16 - base, 0, R - 1)
                for i in range(16):
                    wsp = jnp.full((16,), wm[i], jnp.float32)
                    e = g * 16 + i
                    dr = dl[i]
                    for u in range(8):
                        plsc.addupdate(
                            acc.at[dr, pl.ds(u * 16, 16)],
                            rows[e, pl.ds(u * 16, 16)] * wsp,
                        )
                return 0
            lax.fori_loop(0, G, group_body, 0)
            return 0
        lax.fori_loop(c_lo, c_hi, chunk_body, 0)

        pltpu.sync_copy(acc, s_hbm.at[t, pl.ds(base, R)])


@functools.cache
def _sc_kernels():
    mesh = plsc.VectorSubcoreMesh(
        core_axis_name="c", subcore_axis_name="s",
        num_cores=NC, num_subcores=NS)
    deg_k = pl.kernel(
        _deg_body,
        out_type=jax.ShapeDtypeStruct((NPAD, 128), jnp.float32),
        mesh=mesh,
        scratch_types=[
            pltpu.VMEM((16,), jnp.int32),
            pltpu.VMEM((K,), jnp.int32),
            pltpu.VMEM((K,), jnp.float32),
            pltpu.VMEM((R, 128), jnp.float32),
        ],
    )
    msg_k = pl.kernel(
        _msg_body,
        out_type=jax.ShapeDtypeStruct((T, NPAD, 128), jnp.float32),
        mesh=mesh,
        scratch_types=[
            pltpu.VMEM((16,), jnp.int32),
            pltpu.VMEM((K,), jnp.int32),
            pltpu.VMEM((K,), jnp.int32),
            pltpu.VMEM((K,), jnp.int32),
            pltpu.VMEM((K,), jnp.float32),
            pltpu.VMEM((K, 128), jnp.float32),
            pltpu.VMEM((R, 128), jnp.float32),
            pltpu.SemaphoreType.DMA,
        ],
    )
    return deg_k, msg_k


def _mm_body(x_ref, w_ref, deg_ref, y_ref):
    dinv = lax.rsqrt(deg_ref[:, 0:1])
    xw = jnp.dot(x_ref[0], w_ref[...], preferred_element_type=jnp.float32)
    y_ref[0] = dinv * xw


def _mm(x, w, deg2d, bn):
    nb = N // bn
    return pl.pallas_call(
        _mm_body,
        grid=(T, nb),
        in_specs=[
            pl.BlockSpec((1, bn, C), lambda t, b: (t, b, 0)),
            pl.BlockSpec((C, H), lambda t, b: (0, 0)),
            pl.BlockSpec((bn, 128), lambda t, b: (b, 0)),
        ],
        out_specs=pl.BlockSpec((1, bn, H), lambda t, b: (t, b, 0)),
        out_shape=jax.ShapeDtypeStruct((T, N, H), jnp.float32),
    )(x, w, deg2d)


def _gru_body(s_ref, y_ref, deg_ref, bg_ref, wih_ref, whh_ref, bih_ref,
              bhh_ref, gamma_ref, beta_ref, out_ref):
    dinv = lax.rsqrt(deg_ref[:, 0:1])
    bg = bg_ref[...]
    wih = wih_ref[...]
    whh = whh_ref[...]
    bih = bih_ref[...]
    bhh = bhh_ref[...]
    gamma = gamma_ref[...]
    beta = beta_ref[...]
    h = jnp.zeros_like(y_ref[0])
    for t in range(T):
        s = dinv * (s_ref[t] + y_ref[t]) + bg
        gi = jnp.dot(s, wih, preferred_element_type=jnp.float32) + bih
        gh = jnp.dot(h, whh, preferred_element_type=jnp.float32) + bhh
        r = jax.nn.sigmoid(gi[:, 0:H] + gh[:, 0:H])
        z = jax.nn.sigmoid(gi[:, H:2 * H] + gh[:, H:2 * H])
        n = jnp.tanh(gi[:, 2 * H:] + r * gh[:, 2 * H:])
        h = (1.0 - z) * n + z * h
        m = jnp.mean(h, axis=1, keepdims=True)
        v = jnp.mean((h - m) * (h - m), axis=1, keepdims=True)
        h = (h - m) * lax.rsqrt(v + 1e-5) * gamma + beta
        out_ref[t] = h
    return


def _gru(s3, y, deg2d, bg, wihT, whhT, bih, bhh, gamma, beta, bn):
    nb = N // bn
    return pl.pallas_call(
        _gru_body,
        grid=(nb,),
        in_specs=[
            pl.BlockSpec((T, bn, H), lambda b: (0, b, 0)),
            pl.BlockSpec((T, bn, H), lambda b: (0, b, 0)),
            pl.BlockSpec((bn, 128), lambda b: (b, 0)),
            pl.BlockSpec((1, H), lambda b: (0, 0)),
            pl.BlockSpec((H, 3 * H), lambda b: (0, 0)),
            pl.BlockSpec((H, 3 * H), lambda b: (0, 0)),
            pl.BlockSpec((1, 3 * H), lambda b: (0, 0)),
            pl.BlockSpec((1, 3 * H), lambda b: (0, 0)),
            pl.BlockSpec((1, H), lambda b: (0, 0)),
            pl.BlockSpec((1, H), lambda b: (0, 0)),
        ],
        out_specs=pl.BlockSpec((T, bn, H), lambda b: (0, b, 0)),
        out_shape=jax.ShapeDtypeStruct((T, N, H), jnp.float32),
    )(s3, y, deg2d, bg, wihT, whhT, bih, bhh, gamma, beta)


def kernel(x_sequence, edge_index, edge_weight, Wg, bg, Wih, Whh, bih, bhh,
           gamma, beta):
    src = edge_index[0]
    dst = edge_index[1]
    order = jnp.argsort(dst)
    srcS = src[order].astype(jnp.int32)
    dstS = dst[order].astype(jnp.int32)
    ewS = edge_weight[order]

    offs = jnp.searchsorted(dstS, jnp.arange(NW + 1, dtype=jnp.int32) * R
                            ).astype(jnp.int32)
    bounds = jnp.zeros((NW, 16), jnp.int32)
    bounds = bounds.at[:, 0].set(offs[:-1])
    bounds = bounds.at[:, 1].set(offs[1:])

    deg_kernel, msg_kernel = _sc_kernels()
    deg2d = deg_kernel(dstS, ewS, bounds)

    bn = 400
    outs_prev = x_sequence
    finals = []
    for l in range(2):
        y = _mm(outs_prev, Wg[l], deg2d, bn)
        yflat = y.reshape(T * N, H)
        s3 = msg_kernel(yflat, srcS, dstS, ewS, bounds)
        outs = _gru(
            s3, y, deg2d,
            bg[l].reshape(1, H),
            jnp.swapaxes(Wih[l], 0, 1), jnp.swapaxes(Whh[l], 0, 1),
            bih[l].reshape(1, 3 * H), bhh[l].reshape(1, 3 * H),
            gamma[l].reshape(1, H), beta[l].reshape(1, H),
            bn,
        )
        finals.append(outs[T - 1])
        outs_prev = outs
    return (finals[1], finals[0], finals[1])


# speedup vs baseline: 4.6334x; 1.1469x over previous
import functools

import jax
import jax.numpy as jnp
from jax import lax
from jax.experimental import pallas as pl
from jax.experimental.pallas import tpu as pltpu
from jax.experimental.pallas import tpu_sc as plsc

N = 10000
E = 320000
C = 128
H = 128
T = 8

NC = 2
NS = 16
NW = NC * NS
R = 320
NPAD = NW * R
K = 256
NCHUNK = (E + K - 1) // K
G = K // 16

def _wid():
    return lax.axis_index("s") * NC + lax.axis_index("c")


def _read_bounds(bounds_hbm, bv, wid):
    pltpu.sync_copy(bounds_hbm.at[wid], bv)
    b = bv[...]
    return b[0], b[1]


def _deg_body(dst_hbm, ew_hbm, bounds_hbm, deg_hbm, bv, dv, wv, acc):
    wid = _wid()
    base = wid * R
    e_lo, e_hi = _read_bounds(bounds_hbm, bv, wid)
    c_lo = e_lo // K
    c_hi = (e_hi + (K - 1)) // K

    lane = lax.iota(jnp.int32, 16)
    one0 = jnp.where(lane == 0, 1.0, 0.0)

    def zero_body(i, _):
        for u in range(8):
            acc[i, pl.ds(u * 16, 16)] = jnp.zeros((16,), jnp.float32)
        return 0
    lax.fori_loop(0, R, zero_body, 0)

    def chunk_body(c, _):
        e0 = c * K
        pltpu.sync_copy(dst_hbm.at[pl.ds(e0, K)], dv)
        pltpu.sync_copy(ew_hbm.at[pl.ds(e0, K)], wv)

        def group_body(g, _):
            p = e0 + g * 16 + lane
            d16 = dv[pl.ds(g * 16, 16)]
            w16 = wv[pl.ds(g * 16, 16)]
            valid = (p >= e_lo) & (p < e_hi)
            wm = jnp.where(valid, w16, 0.0)
            dl = jnp.clip(d16 - base, 0, R - 1)
            for i in range(16):
                plsc.addupdate(acc.at[dl[i], pl.ds(0, 16)], one0 * wm[i])
            return 0
        lax.fori_loop(0, G, group_body, 0)
        return 0
    lax.fori_loop(c_lo, c_hi, chunk_body, 0)

    def self_body(i, _):
        plsc.addupdate(acc.at[i, pl.ds(0, 16)], one0)
        return 0
    lax.fori_loop(0, R, self_body, 0)

    pltpu.sync_copy(acc, deg_hbm.at[pl.ds(base, R)])


def _msg_body(yflat_hbm, src_hbm, dst_hbm, ew_hbm, bounds_hbm, s_hbm,
              bv, sv, gv0, gv1, dv0, dv1, wv0, wv1, rows0, rows1,
              acc, sem0, sem1):
    wid = _wid()
    base = wid * R
    e_lo, e_hi = _read_bounds(bounds_hbm, bv, wid)
    c_lo = e_lo // K
    c_hi = (e_hi + (K - 1)) // K

    lane = lax.iota(jnp.int32, 16)
    bufs = ((gv0, dv0, wv0, rows0, sem0), (gv1, dv1, wv1, rows1, sem1))

    def t_body(t, _):
        def zero_body(i, _):
            for u in range(8):
                acc[i, pl.ds(u * 16, 16)] = jnp.zeros((16,), jnp.float32)
            return 0
        lax.fori_loop(0, R, zero_body, 0)

        def start(c, gv, dv, wv, rows, sem):
            e0 = c * K
            pltpu.sync_copy(src_hbm.at[pl.ds(e0, K)], sv)
            pltpu.sync_copy(dst_hbm.at[pl.ds(e0, K)], dv)
            pltpu.sync_copy(ew_hbm.at[pl.ds(e0, K)], wv)

            def prep_body(g, _):
                o = g * 16
                gv[pl.ds(o, 16)] = sv[pl.ds(o, 16)] + t * N
                p = e0 + o + lane
                valid = (p >= e_lo) & (p < e_hi)
                wv[pl.ds(o, 16)] = jnp.where(valid, wv[pl.ds(o, 16)], 0.0)
                dv[pl.ds(o, 16)] = jnp.clip(dv[pl.ds(o, 16)] - base, 0, R - 1)
                return 0
            lax.fori_loop(0, G, prep_body, 0)
            pltpu.async_copy(yflat_hbm.at[gv], rows, sem)

        def process(c, gv, dv, wv, rows, sem):
            pltpu.make_async_copy(yflat_hbm.at[gv], rows, sem).wait()

            def group_body(g, _):
                dl = dv[pl.ds(g * 16, 16)]
                wm = wv[pl.ds(g * 16, 16)]
                for i in range(16):
                    wsp = jnp.full((16,), wm[i], jnp.float32)
                    e = g * 16 + i
                    dr = dl[i]
                    for u in range(8):
                        plsc.addupdate(
                            acc.at[dr, pl.ds(u * 16, 16)],
                            rows[e, pl.ds(u * 16, 16)] * wsp,
                        )
                return 0
            lax.fori_loop(0, G, group_body, 0)

        @pl.when(c_lo < c_hi)
        def _():
            start(c_lo, *bufs[0])

        @pl.when(c_lo + 1 < c_hi)
        def _():
            start(c_lo + 1, *bufs[1])

        def pair_body(k, _):
            c0 = c_lo + 2 * k
            for b in range(2):
                c = c0 + b

                @pl.when(c < c_hi)
                def _():
                    process(c, *bufs[b])

                    @pl.when(c + 2 < c_hi)
                    def _():
                        start(c + 2, *bufs[b])
            return 0
        npairs = (c_hi - c_lo + 1) // 2
        lax.fori_loop(0, npairs, pair_body, 0)

        pltpu.sync_copy(acc, s_hbm.at[t, pl.ds(base, R)])
        return 0

    lax.fori_loop(0, T, t_body, 0)


@functools.cache
def _sc_kernels():
    mesh = plsc.VectorSubcoreMesh(
        core_axis_name="c", subcore_axis_name="s",
        num_cores=NC, num_subcores=NS)
    deg_k = pl.kernel(
        _deg_body,
        out_type=jax.ShapeDtypeStruct((NPAD, 128), jnp.float32),
        mesh=mesh,
        scratch_types=[
            pltpu.VMEM((16,), jnp.int32),
            pltpu.VMEM((K,), jnp.int32),
            pltpu.VMEM((K,), jnp.float32),
            pltpu.VMEM((R, 128), jnp.float32),
        ],
    )
    msg_k = pl.kernel(
        _msg_body,
        out_type=jax.ShapeDtypeStruct((T, NPAD, 128), jnp.float32),
        mesh=mesh,
        scratch_types=[
            pltpu.VMEM((16,), jnp.int32),
            pltpu.VMEM((K,), jnp.int32),
            pltpu.VMEM((K,), jnp.int32),
            pltpu.VMEM((K,), jnp.int32),
            pltpu.VMEM((K,), jnp.int32),
            pltpu.VMEM((K,), jnp.int32),
            pltpu.VMEM((K,), jnp.float32),
            pltpu.VMEM((K,), jnp.float32),
            pltpu.VMEM((K, 128), jnp.float32),
            pltpu.VMEM((K, 128), jnp.float32),
            pltpu.VMEM((R, 128), jnp.float32),
            pltpu.SemaphoreType.DMA,
            pltpu.SemaphoreType.DMA,
        ],
    )
    return deg_k, msg_k


def _mm_body(x_ref, w_ref, deg_ref, y_ref):
    dinv = lax.rsqrt(deg_ref[:, 0:1])
    xw = jnp.dot(x_ref[0], w_ref[...], preferred_element_type=jnp.float32)
    y_ref[0] = dinv * xw


def _mm(x, w, deg2d, bn):
    nb = N // bn
    return pl.pallas_call(
        _mm_body,
        grid=(T, nb),
        in_specs=[
            pl.BlockSpec((1, bn, C), lambda t, b: (t, b, 0)),
            pl.BlockSpec((C, H), lambda t, b: (0, 0)),
            pl.BlockSpec((bn, 128), lambda t, b: (b, 0)),
        ],
        out_specs=pl.BlockSpec((1, bn, H), lambda t, b: (t, b, 0)),
        out_shape=jax.ShapeDtypeStruct((T, N, H), jnp.float32),
    )(x, w, deg2d)


def _gru_body(s_ref, y_ref, deg_ref, bg_ref, wih_ref, whh_ref, bih_ref,
              bhh_ref, gamma_ref, beta_ref, out_ref):
    dinv = lax.rsqrt(deg_ref[:, 0:1])
    bg = bg_ref[...]
    wih = wih_ref[...]
    whh = whh_ref[...]
    bih = bih_ref[...]
    bhh = bhh_ref[...]
    gamma = gamma_ref[...]
    beta = beta_ref[...]
    h = jnp.zeros_like(y_ref[0])
    for t in range(T):
        s = dinv * (s_ref[t] + y_ref[t]) + bg
        gi = jnp.dot(s, wih, preferred_element_type=jnp.float32) + bih
        gh = jnp.dot(h, whh, preferred_element_type=jnp.float32) + bhh
        r = jax.nn.sigmoid(gi[:, 0:H] + gh[:, 0:H])
        z = jax.nn.sigmoid(gi[:, H:2 * H] + gh[:, H:2 * H])
        n = jnp.tanh(gi[:, 2 * H:] + r * gh[:, 2 * H:])
        h = (1.0 - z) * n + z * h
        m = jnp.mean(h, axis=1, keepdims=True)
        v = jnp.mean((h - m) * (h - m), axis=1, keepdims=True)
        h = (h - m) * lax.rsqrt(v + 1e-5) * gamma + beta
        out_ref[t] = h
    return


def _gru(s3, y, deg2d, bg, wihT, whhT, bih, bhh, gamma, beta, bn):
    nb = N // bn
    return pl.pallas_call(
        _gru_body,
        grid=(nb,),
        in_specs=[
            pl.BlockSpec((T, bn, H), lambda b: (0, b, 0)),
            pl.BlockSpec((T, bn, H), lambda b: (0, b, 0)),
            pl.BlockSpec((bn, 128), lambda b: (b, 0)),
            pl.BlockSpec((1, H), lambda b: (0, 0)),
            pl.BlockSpec((H, 3 * H), lambda b: (0, 0)),
            pl.BlockSpec((H, 3 * H), lambda b: (0, 0)),
            pl.BlockSpec((1, 3 * H), lambda b: (0, 0)),
            pl.BlockSpec((1, 3 * H), lambda b: (0, 0)),
            pl.BlockSpec((1, H), lambda b: (0, 0)),
            pl.BlockSpec((1, H), lambda b: (0, 0)),
        ],
        out_specs=pl.BlockSpec((T, bn, H), lambda b: (0, b, 0)),
        out_shape=jax.ShapeDtypeStruct((T, N, H), jnp.float32),
    )(s3, y, deg2d, bg, wihT, whhT, bih, bhh, gamma, beta)


def kernel(x_sequence, edge_index, edge_weight, Wg, bg, Wih, Whh, bih, bhh,
           gamma, beta):
    src = edge_index[0]
    dst = edge_index[1]
    order = jnp.argsort(dst)
    srcS = src[order].astype(jnp.int32)
    dstS = dst[order].astype(jnp.int32)
    ewS = edge_weight[order]

    offs = jnp.searchsorted(dstS, jnp.arange(NW + 1, dtype=jnp.int32) * R
                            ).astype(jnp.int32)
    bounds = jnp.zeros((NW, 16), jnp.int32)
    bounds = bounds.at[:, 0].set(offs[:-1])
    bounds = bounds.at[:, 1].set(offs[1:])

    deg_kernel, msg_kernel = _sc_kernels()
    deg2d = deg_kernel(dstS, ewS, bounds)

    bn = 400
    outs_prev = x_sequence
    finals = []
    for l in range(2):
        y = _mm(outs_prev, Wg[l], deg2d, bn)
        yflat = y.reshape(T * N, H)
        s3 = msg_kernel(yflat, srcS, dstS, ewS, bounds)
        outs = _gru(
            s3, y, deg2d,
            bg[l].reshape(1, H),
            jnp.swapaxes(Wih[l], 0, 1), jnp.swapaxes(Whh[l], 0, 1),
            bih[l].reshape(1, 3 * H), bhh[l].reshape(1, 3 * H),
            gamma[l].reshape(1, H), beta[l].reshape(1, H),
            bn,
        )
        finals.append(outs[T - 1])
        outs_prev = outs
    return (finals[1], finals[0], finals[1])
